# baseline (speedup 1.0000x reference)
import jax
import jax.numpy as jnp
from jax import lax
from jax.experimental import pallas as pl
from jax.experimental.pallas import tpu as pltpu
from jax.experimental.pallas import tpu_sc as plsc

K = 129
D_MODEL = 1024
NUM_HEADS = 16
SCALE = 32.0

HALF0 = 64
HALF1 = K - HALF0
RPS = 8


def _scale_rows(buf, nrows):
    for r in range(nrows):
        def body(i, carry, r=r):
            sl = pl.ds(i * 16, 16)
            buf[r, sl] = buf[r, sl] * SCALE
            return carry
        lax.fori_loop(0, D_MODEL // 16, body, 0)


def _body(wq, wk, wv, oq, ok, ov, sh0, sh1, sh2, buf, sem):
    s = lax.axis_index("s")
    c = lax.axis_index("c")
    shared = (sh0, sh1, sh2)
    tabs = (wq, wk, wv)
    outs = (oq, ok, ov)

    @pl.when(s < 8)
    def _stage():
        base = c * HALF0 + s * RPS
        for t in range(3):
            pltpu.sync_copy(tabs[t].at[pl.ds(base, RPS)], buf)
            _scale_rows(buf, RPS)
            pltpu.sync_copy(buf, shared[t].at[pl.ds(s * RPS, RPS)])

    plsc.subcore_barrier()

    NCH = 4
    CH = HALF0 // NCH

    def _chunked(base_out):
        descs = []
        for t in range(3):
            for j in range(NCH):
                off = pl.multiple_of(((s + j) % NCH) * CH, CH)
                descs.append(pltpu.async_copy(
                    shared[t].at[pl.ds(off, CH)],
                    outs[t].at[0, s, pl.ds(pl.multiple_of(base_out + off, CH),
                                           CH)],
                    sem))
        return descs

    @pl.when(c == 0)
    def _lo():
        for d in _chunked(0):
            d.wait()

    @pl.when(c == 1)
    def _hi():
        descs = _chunked(HALF0)
        for t in range(3):
            pltpu.sync_copy(tabs[t].at[pl.ds(K - 1, 1)], buf.at[pl.ds(0, 1)])
            _scale_rows(buf, 1)
            pltpu.sync_copy(buf.at[pl.ds(0, 1)],
                            outs[t].at[0, s, pl.ds(K - 1, 1)])
        for d in descs:
            d.wait()


def kernel(Wq, Wk, Wv):
    out = jax.ShapeDtypeStruct((1, NUM_HEADS, K, D_MODEL), jnp.float32)
    mesh = plsc.VectorSubcoreMesh(core_axis_name="c", subcore_axis_name="s")
    f = pl.kernel(
        _body,
        out_type=[out, out, out],
        mesh=mesh,
        scratch_types=[
            pltpu.VMEM_SHARED((HALF0, D_MODEL), jnp.float32),
            pltpu.VMEM_SHARED((HALF0, D_MODEL), jnp.float32),
            pltpu.VMEM_SHARED((HALF0, D_MODEL), jnp.float32),
            pltpu.VMEM((RPS, D_MODEL), jnp.float32),
            pltpu.SemaphoreType.DMA,
        ],
    )
    return tuple(f(Wq, Wk, Wv))


# speedup vs baseline: 1.2584x; 1.2584x over previous
import jax
import jax.numpy as jnp
from jax import lax
from jax.experimental import pallas as pl
from jax.experimental.pallas import tpu as pltpu
from jax.experimental.pallas import tpu_sc as plsc

K = 129
D_MODEL = 1024
NUM_HEADS = 16
SCALE = 32.0

HALF0 = 64
RPS = 8


def _scale_rows(buf, nrows):
    for r in range(nrows):
        def body(i, carry, r=r):
            sl = pl.ds(i * 16, 16)
            buf[r, sl] = buf[r, sl] * SCALE
            return carry
        lax.fori_loop(0, D_MODEL // 16, body, 0)


def _sc_body(wv, ov, shared, buf, sem):
    s = lax.axis_index("s")
    c = lax.axis_index("c")

    @pl.when(s < 8)
    def _stage():
        base = c * HALF0 + s * RPS
        pltpu.sync_copy(wv.at[pl.ds(base, RPS)], buf)
        _scale_rows(buf, RPS)
        pltpu.sync_copy(buf, shared.at[pl.ds(s * RPS, RPS)])

    plsc.subcore_barrier()

    @pl.when(c == 0)
    def _lo():
        pltpu.async_copy(shared.at[pl.ds(0, HALF0)],
                         ov.at[0, s, pl.ds(0, HALF0)], sem).wait()

    @pl.when(c == 1)
    def _hi():
        d = pltpu.async_copy(shared.at[pl.ds(0, HALF0)],
                             ov.at[0, s, pl.ds(HALF0, HALF0)], sem)
        pltpu.sync_copy(wv.at[pl.ds(K - 1, 1)], buf.at[pl.ds(0, 1)])
        _scale_rows(buf, 1)
        pltpu.sync_copy(buf.at[pl.ds(0, 1)], ov.at[0, s, pl.ds(K - 1, 1)])
        d.wait()


def _sc_rel_v(Wv):
    out = jax.ShapeDtypeStruct((1, NUM_HEADS, K, D_MODEL), jnp.float32)
    mesh = plsc.VectorSubcoreMesh(core_axis_name="c", subcore_axis_name="s")
    f = pl.kernel(
        _sc_body,
        out_type=out,
        mesh=mesh,
        scratch_types=[
            pltpu.VMEM_SHARED((HALF0, D_MODEL), jnp.float32),
            pltpu.VMEM((RPS, D_MODEL), jnp.float32),
            pltpu.SemaphoreType.DMA,
        ],
    )
    return f(Wv)


HEADS_PER_STEP = 4


def _tc_body(wq_ref, wk_ref, oq_ref, ok_ref):
    for o_ref, w_ref in ((oq_ref, wq_ref), (ok_ref, wk_ref)):
        w = w_ref[...] * SCALE
        o_ref[0] = jnp.broadcast_to(w[None], (HEADS_PER_STEP, K, D_MODEL))


def _tc_rel_qk(Wq, Wk):
    in_spec = pl.BlockSpec((K, D_MODEL), lambda h: (0, 0))
    out_spec = pl.BlockSpec(
        (1, HEADS_PER_STEP, K, D_MODEL), lambda h: (0, h, 0, 0)
    )
    out_shape = jax.ShapeDtypeStruct((1, NUM_HEADS, K, D_MODEL), jnp.float32)
    return pl.pallas_call(
        _tc_body,
        grid=(NUM_HEADS // HEADS_PER_STEP,),
        in_specs=[in_spec, in_spec],
        out_specs=[out_spec, out_spec],
        out_shape=[out_shape, out_shape],
        compiler_params=pltpu.CompilerParams(
            dimension_semantics=("parallel",)
        ),
    )(Wq, Wk)


def kernel(Wq, Wk, Wv):
    rel_v = _sc_rel_v(Wv)
    rel_q, rel_k = _tc_rel_qk(Wq, Wk)
    return (rel_q, rel_k, rel_v)


# speedup vs baseline: 1.9192x; 1.5251x over previous
import jax
import jax.numpy as jnp
from jax.experimental import pallas as pl
from jax.experimental.pallas import tpu as pltpu

K = 129
D_MODEL = 1024
NUM_HEADS = 16
SCALE = 32.0
NSEM = 8


def _body(wq_hbm, wk_hbm, wv_hbm, oq_hbm, ok_hbm, ov_hbm,
          vq, vk, vv, sem_in, *sems):
    pairs = ((wq_hbm, vq), (wk_hbm, vk), (wv_hbm, vv))
    in_copies = [pltpu.make_async_copy(src, dst, sem_in) for src, dst in pairs]
    for c in in_copies:
        c.start()
    for c in in_copies:
        c.wait()
    vq[...] = vq[...] * SCALE
    vk[...] = vk[...] * SCALE
    vv[...] = vv[...] * SCALE

    out_copies = []
    i = 0
    for v, o in ((vq, oq_hbm), (vk, ok_hbm), (vv, ov_hbm)):
        for h in range(NUM_HEADS):
            out_copies.append(
                pltpu.make_async_copy(v, o.at[0, h], sems[i % NSEM])
            )
            i += 1
    for c in out_copies:
        c.start()
    for c in out_copies:
        c.wait()


def kernel(Wq, Wk, Wv):
    any_spec = pl.BlockSpec(memory_space=pltpu.MemorySpace.HBM)
    out_shape = jax.ShapeDtypeStruct((1, NUM_HEADS, K, D_MODEL), jnp.float32)
    return tuple(pl.pallas_call(
        _body,
        in_specs=[any_spec] * 3,
        out_specs=[any_spec] * 3,
        out_shape=[out_shape] * 3,
        scratch_shapes=[
            pltpu.VMEM((K, D_MODEL), jnp.float32),
            pltpu.VMEM((K, D_MODEL), jnp.float32),
            pltpu.VMEM((K, D_MODEL), jnp.float32),
            pltpu.SemaphoreType.DMA,
        ] + [pltpu.SemaphoreType.DMA] * NSEM,
    )(Wq, Wk, Wv))
